# baseline (speedup 1.0000x reference)
import functools

import jax
import jax.numpy as jnp
from jax import lax
from jax.experimental import pallas as pl
from jax.experimental.pallas import tpu as pltpu

_C = 100000
_D = 128
_B = 16384
_Q = 1024
_CBLK = 2048


def _mm_body(w_ref, q_ref, out_ref, qn_ref):
    @pl.when(pl.program_id(0) == 0)
    def _():
        q = q_ref[...]
        qs = jnp.sum(q * q, axis=1, keepdims=True)
        qn = q / jnp.maximum(jnp.sqrt(qs), 1e-8)
        qn_ref[...] = qn.astype(jnp.bfloat16)

    w = w_ref[...]
    s = jnp.sum(w * w, axis=1, keepdims=True)
    wn = (w / jnp.maximum(jnp.sqrt(s), 1e-8)).astype(jnp.bfloat16)
    out_ref[...] = lax.dot_general(
        qn_ref[...], wn, (((1,), (1,)), ((), ())),
        preferred_element_type=jnp.float32)


def _cosine_logits(new_weight, query):
    return pl.pallas_call(
        _mm_body,
        grid=((_C + _CBLK - 1) // _CBLK,),
        in_specs=[
            pl.BlockSpec((_CBLK, _D), lambda i: (i, 0)),
            pl.BlockSpec((_Q, _D), lambda i: (0, 0)),
        ],
        out_specs=pl.BlockSpec((_Q, _CBLK), lambda i: (0, i)),
        out_shape=jax.ShapeDtypeStruct((_Q, _C), jnp.float32),
        scratch_shapes=[pltpu.VMEM((_Q, _D), jnp.bfloat16)],
    )(new_weight, query)


def kernel(weight, input, idx, query):
    new_weight = weight.at[idx].set(weight[idx] + input)
    return _cosine_logits(new_weight, query)
